# Initial kernel scaffold; baseline (speedup 1.0000x reference)
import jax
import jax.numpy as jnp
from jax.experimental import pallas as pl


def kernel(x_in, edge_index, gamma0, beta0, mean0, var0, W_gcn, b_gcn, gamma1, beta1, mean1, var1, W_skip, b_skip, W_gat, att_src, att_dst, b_gat, gamma2, beta2, mean2, var2, W_px1, b_px1, skip_weight, alpha_mix, W_np1, b_np1, W_np2, b_np2, W_np3, b_np3):
    raise NotImplementedError("write your pallas kernel here")


# speedup vs baseline: 8.7101x; 8.7101x over previous
import functools
import jax
import jax.numpy as jnp
from jax import lax
from jax.experimental import pallas as pl
from jax.experimental.pallas import tpu as pltpu
from jax.experimental.pallas import tpu_sc as plsc

F32 = jnp.float32

N_NODES = 10000
N_EDGES = 320000
IND = 128
OUTD = 128
HEADS = 5
HID = 2 * OUTD

NPAD = 10240
NC = 2
NS = 16
NW = NC * NS
EPT = N_EDGES // NW
K = 80
ITERS = EPT // K
RPT = NPAD // NS

_sc_mesh = plsc.VectorSubcoreMesh(core_axis_name="c", subcore_axis_name="s")

HNP = NPAD // 2
ACC_R = HNP + 16
ZPT = ACC_R // NS
RPT2 = HNP // NS
EPT2 = N_EDGES // NS
ITERS2 = EPT2 // K


def _zero16(ref, nrows, ncol16):
    z = jnp.zeros((16,), F32)

    def body(j, _):
        for q in range(ncol16):
            ref[j, pl.ds(q * 16, 16)] = z
        return 0

    lax.fori_loop(0, nrows, body, 0)


@functools.partial(
    pl.kernel,
    out_type=jax.ShapeDtypeStruct((NW, NPAD), F32),
    mesh=_sc_mesh,
    scratch_types=[
        pltpu.VMEM((K,), jnp.int32),
        pltpu.VMEM((NPAD,), F32),
    ],
    compiler_params=pltpu.CompilerParams(needs_layout_passes=False),
)
def _sc_deg(dst_hbm, deg_out, didx, deg_t):
    c = lax.axis_index("c")
    s = lax.axis_index("s")
    w = c * NS + s
    z = jnp.zeros((16,), F32)

    def zr(j, _):
        deg_t[pl.ds(j * 16, 16)] = z
        return 0

    lax.fori_loop(0, NPAD // 16, zr, 0)

    ones = jnp.ones((16,), F32)
    woff = w * EPT

    def step(i, _):
        base = pl.multiple_of(woff + i * K, 8)
        pltpu.sync_copy(dst_hbm.at[pl.ds(base, K)], didx)

        def grp(g, _):
            d16 = didx[pl.ds(g * 16, 16)]
            plsc.addupdate_scatter(deg_t, [d16], ones)
            return 0

        lax.fori_loop(0, K // 16, grp, 0)
        return 0

    lax.fori_loop(0, ITERS, step, 0)
    pltpu.sync_copy(deg_t, deg_out.at[w])


def _localize(didx, lidx, c):
    lo = c * HNP

    def body(jj, _):
        v = didx[pl.ds(jj * 16, 16)] - lo
        ok = jnp.logical_and(v >= 0, v < HNP)
        lidx[pl.ds(jj * 16, 16)] = jnp.where(ok, v, HNP)
        return 0

    lax.fori_loop(0, K // 16, body, 0)


@functools.partial(
    pl.kernel,
    out_type=jax.ShapeDtypeStruct((NC, HNP, 128), F32),
    mesh=_sc_mesh,
    scratch_types=[
        pltpu.VMEM((K,), jnp.int32),
        pltpu.VMEM((K,), jnp.int32),
        pltpu.VMEM((K,), jnp.int32),
        pltpu.VMEM((K, 128), F32),
        pltpu.VMEM((ZPT, 128), F32),
        pltpu.SemaphoreType.DMA,
        pltpu.VMEM_SHARED((ACC_R, 128), F32),
    ],
)
def _sc_gcn(src_hbm, dst_hbm, xls_hbm, agg_out, sidx, didx, lidx, rows, stage,
            sem, agg_sh):
    c = lax.axis_index("c")
    s = lax.axis_index("s")

    _zero16(stage, ZPT, 8)
    pltpu.sync_copy(stage, agg_sh.at[pl.ds(s * ZPT, ZPT)])
    plsc.subcore_barrier()

    woff = s * EPT2

    def step(i, _):
        base = pl.multiple_of(woff + i * K, 8)
        pltpu.sync_copy(src_hbm.at[pl.ds(base, K)], sidx)
        pltpu.sync_copy(dst_hbm.at[pl.ds(base, K)], didx)
        _localize(didx, lidx, c)
        pltpu.async_copy(xls_hbm.at[sidx], rows, sem).wait()
        pltpu.sync_copy(rows, agg_sh.at[lidx], add=True)
        return 0

    lax.fori_loop(0, ITERS2, step, 0)
    plsc.subcore_barrier()

    pltpu.sync_copy(agg_sh.at[pl.ds(s * RPT2, RPT2)],
                    stage.at[pl.ds(0, RPT2)])
    pltpu.sync_copy(stage.at[pl.ds(0, RPT2)],
                    agg_out.at[c, pl.ds(s * RPT2, RPT2)])


@functools.partial(
    pl.kernel,
    out_type=jax.ShapeDtypeStruct((N_EDGES * 16,), F32),
    mesh=_sc_mesh,
    scratch_types=[
        pltpu.VMEM((K,), jnp.int32),
        pltpu.VMEM((K,), jnp.int32),
        pltpu.VMEM((N_NODES * HEADS,), F32),
        pltpu.VMEM((N_NODES * HEADS,), F32),
        pltpu.VMEM((K * 16,), F32),
    ],
    compiler_params=pltpu.CompilerParams(needs_layout_passes=False),
)
def _sc_att(src_hbm, dst_hbm, asf_hbm, adf_hbm, exb_out, sidx, didx,
            as_t, ad_t, exv):
    c = lax.axis_index("c")
    s = lax.axis_index("s")

    pltpu.sync_copy(asf_hbm, as_t)
    pltpu.sync_copy(adf_hbm, ad_t)
    z = jnp.zeros((16,), F32)

    def zr(j, _):
        exv[pl.ds(j * 16, 16)] = z
        return 0

    lax.fori_loop(0, K, zr, 0)

    woff = (c * NS + s) * EPT

    def step(i, _):
        base = pl.multiple_of(woff + i * K, 8)
        pltpu.sync_copy(src_hbm.at[pl.ds(base, K)], sidx)
        pltpu.sync_copy(dst_hbm.at[pl.ds(base, K)], didx)

        def grp(g, _):
            s16 = sidx[pl.ds(g * 16, 16)] * HEADS
            d16 = didx[pl.ds(g * 16, 16)] * HEADS
            eids = (jnp.arange(16, dtype=jnp.int32) + g * 16) * 16
            for h in range(HEADS):
                av = plsc.load_gather(as_t, [s16 + h])
                dv = plsc.load_gather(ad_t, [d16 + h])
                v = av + dv
                v = jnp.where(v > 0, v, 0.2 * v)
                plsc.store_scatter(exv, [eids + h], jnp.exp(v))
            return 0

        lax.fori_loop(0, K // 16, grp, 0)
        pltpu.sync_copy(exv, exb_out.at[pl.ds(base * 16, K * 16)])
        return 0

    lax.fori_loop(0, ITERS, step, 0)


@functools.partial(
    pl.kernel,
    out_type=jax.ShapeDtypeStruct((NW, NPAD * 8), F32),
    mesh=_sc_mesh,
    scratch_types=[
        pltpu.VMEM((K,), jnp.int32),
        pltpu.VMEM((K * 16,), F32),
        pltpu.VMEM((NPAD * 8,), F32),
    ],
    compiler_params=pltpu.CompilerParams(needs_layout_passes=False),
)
def _sc_den(dst_hbm, exb_hbm, den_out, didx, exv, den_t):
    c = lax.axis_index("c")
    s = lax.axis_index("s")
    w = c * NS + s
    z = jnp.zeros((16,), F32)

    def zr(j, _):
        den_t[pl.ds(j * 16, 16)] = z
        return 0

    lax.fori_loop(0, NPAD * 8 // 16, zr, 0)

    woff = w * EPT

    def step(i, _):
        base = pl.multiple_of(woff + i * K, 8)
        pltpu.sync_copy(dst_hbm.at[pl.ds(base, K)], didx)
        pltpu.sync_copy(exb_hbm.at[pl.ds(base * 16, K * 16)], exv)

        def grp(g, _):
            d16 = didx[pl.ds(g * 16, 16)] * 8
            eids = (jnp.arange(16, dtype=jnp.int32) + g * 16) * 16
            for h in range(HEADS):
                exg = plsc.load_gather(exv, [eids + h])
                plsc.addupdate_scatter(den_t, [d16 + h], exg)
            return 0

        lax.fori_loop(0, K // 16, grp, 0)
        return 0

    lax.fori_loop(0, ITERS, step, 0)
    pltpu.sync_copy(den_t, den_out.at[w])


@functools.partial(
    pl.kernel,
    out_type=jax.ShapeDtypeStruct((NC, HEADS, HNP, 128), F32),
    mesh=_sc_mesh,
    scratch_types=[
        pltpu.VMEM((K,), jnp.int32),
        pltpu.VMEM((K,), jnp.int32),
        pltpu.VMEM((K,), jnp.int32),
        pltpu.VMEM((K, 128), F32),
        pltpu.VMEM((K, 128), F32),
        pltpu.VMEM((K, 16), F32),
        pltpu.VMEM((ZPT, 128), F32),
        pltpu.SemaphoreType.DMA,
        pltpu.VMEM_SHARED((ACC_R, 128), F32),
    ],
)
def _sc_gat(src_hbm, dst_hbm, skip1_hbm, exb_hbm, agg_out, sidx,
            didx, lidx, rows, wrows, exv, stage, sem, agg_sh):
    c = lax.axis_index("c")
    s = lax.axis_index("s")
    woff = s * EPT2

    for h in range(HEADS):
        _zero16(stage, ZPT, 8)
        pltpu.sync_copy(stage, agg_sh.at[pl.ds(s * ZPT, ZPT)])
        plsc.subcore_barrier()

        def step(i, _):
            base = pl.multiple_of(woff + i * K, 8)
            pltpu.sync_copy(src_hbm.at[pl.ds(base, K)], sidx)
            pltpu.sync_copy(dst_hbm.at[pl.ds(base, K)], didx)
            _localize(didx, lidx, c)
            pltpu.sync_copy(exb_hbm.at[pl.ds(base, K)], exv)
            pltpu.async_copy(skip1_hbm.at[sidx], rows, sem).wait()

            def inner(j, _):
                ev = exv[j, :]
                w = jnp.full((16,), ev[h], F32)
                for q in range(8):
                    wrows[j, pl.ds(q * 16, 16)] = (
                        rows[j, pl.ds(q * 16, 16)] * w)
                return 0

            lax.fori_loop(0, K, inner, 0)
            pltpu.sync_copy(wrows, agg_sh.at[lidx], add=True)
            return 0

        lax.fori_loop(0, ITERS2, step, 0)
        plsc.subcore_barrier()

        pltpu.sync_copy(agg_sh.at[pl.ds(s * RPT2, RPT2)],
                        stage.at[pl.ds(0, RPT2)])
        pltpu.sync_copy(stage.at[pl.ds(0, RPT2)],
                        agg_out.at[c, h, pl.ds(s * RPT2, RPT2)])
        plsc.subcore_barrier()


RB = 512
GRID = NPAD // RB


def _full(shape):
    return pl.BlockSpec(shape, lambda i: tuple(0 for _ in shape))


def _rows(width, lead=()):
    nlead = len(lead)
    return pl.BlockSpec(lead + (RB, width),
                        lambda i: tuple(0 for _ in range(nlead)) + (i, 0))


def _leaky(x, slope=0.01):
    return jnp.where(x > 0, x, slope * x)


def _bnk(x, g, b, m, v):
    return (x - m) * lax.rsqrt(v + 1e-5) * g + b


def _tc_prep_body(W_gat, att_s, att_d, As_ref, Ad_ref):
    cols_s = []
    cols_d = []
    for h in range(HEADS):
        Wh = W_gat[:, h * HID:(h + 1) * HID]
        cols_s.append(jnp.dot(Wh, att_s[h, :][:, None],
                              preferred_element_type=F32))
        cols_d.append(jnp.dot(Wh, att_d[h, :][:, None],
                              preferred_element_type=F32))
    z = jnp.zeros((IND, 8 - HEADS), F32)
    As_ref[...] = jnp.concatenate(cols_s + [z], axis=1)
    Ad_ref[...] = jnp.concatenate(cols_d + [z], axis=1)


def _tc_prep(W_gat, att_src, att_dst):
    return pl.pallas_call(
        _tc_prep_body,
        grid=(1,),
        in_specs=[_full((IND, HEADS * HID)), _full((HEADS, HID)),
                  _full((HEADS, HID))],
        out_specs=[_full((IND, 8)), _full((IND, 8))],
        out_shape=[jax.ShapeDtypeStruct((IND, 8), F32),
                   jax.ShapeDtypeStruct((IND, 8), F32)],
    )(W_gat, att_src, att_dst)


def _tc1_body(xin, g0, b0, m0, v0, Wg, Ws, bs, xl_ref, xp_ref):
    x = _bnk(xin[...], g0[...], b0[...], m0[...], v0[...])
    xl_ref[...] = jnp.dot(x, Wg[...], preferred_element_type=F32)
    xp_ref[...] = jnp.dot(x, Ws[...], preferred_element_type=F32) + bs[...]


def _tc1(x_in, g0, b0, m0, v0, W_gcn, W_skip, b_skip):
    return pl.pallas_call(
        _tc1_body,
        grid=(GRID,),
        in_specs=[_rows(IND)] + [_full((1, IND))] * 4 +
                 [_full((IND, OUTD)), _full((IND, OUTD)), _full((1, OUTD))],
        out_specs=[_rows(OUTD), _rows(OUTD)],
        out_shape=[jax.ShapeDtypeStruct((NPAD, OUTD), F32),
                   jax.ShapeDtypeStruct((NPAD, OUTD), F32)],
    )(x_in, g0, b0, m0, v0, W_gcn, W_skip, b_skip)


def _tc2_body(degp, xl, xls_ref):
    deg = jnp.sum(degp[...], axis=0)[:, None] + 1.0
    dis = lax.rsqrt(deg)
    xls_ref[...] = dis * xl[...]


def _tc2(deg_p, xl):
    return pl.pallas_call(
        _tc2_body,
        grid=(GRID,),
        in_specs=[pl.BlockSpec((NW, RB), lambda i: (0, i)), _rows(OUTD)],
        out_specs=_rows(OUTD),
        out_shape=jax.ShapeDtypeStruct((NPAD, OUTD), F32),
    )(deg_p, xl)


def _tc3_body(degp, aggp, xl, xp, g1, b1, m1, v1, bg, As, Ad, sw,
              x1_ref, skip1_ref, as_ref, ad_ref):
    deg = jnp.sum(degp[...], axis=0)[:, None] + 1.0
    dis = lax.rsqrt(deg)
    tot = aggp[0] + dis * xl[...]
    x1 = dis * tot + bg[...]
    x1 = _leaky(_bnk(x1, g1[...], b1[...], m1[...], v1[...]))
    skip1 = sw[0, 0] * xp[...] + x1
    x1_ref[...] = x1
    skip1_ref[...] = skip1
    as_ref[...] = jnp.dot(skip1, As[...], preferred_element_type=F32)
    ad_ref[...] = jnp.dot(skip1, Ad[...], preferred_element_type=F32)


def _tc3(deg_p, agg_p, xl, xp, g1, b1, m1, v1, b_gcn, As, Ad, sw):
    return pl.pallas_call(
        _tc3_body,
        grid=(GRID,),
        in_specs=[pl.BlockSpec((NW, RB), lambda i: (0, i)),
                  pl.BlockSpec((1, RB, OUTD),
                               lambda i: (i // (HNP // RB), i % (HNP // RB),
                                          0)),
                  _rows(OUTD), _rows(OUTD)] + [_full((1, OUTD))] * 5 +
                 [_full((OUTD, 8)), _full((OUTD, 8)), _full((1, 1))],
        out_specs=[_rows(OUTD), _rows(OUTD), _rows(8), _rows(8)],
        out_shape=[jax.ShapeDtypeStruct((NPAD, OUTD), F32),
                   jax.ShapeDtypeStruct((NPAD, OUTD), F32),
                   jax.ShapeDtypeStruct((NPAD, 8), F32),
                   jax.ShapeDtypeStruct((NPAD, 8), F32)],
    )(deg_p, agg_p, xl, xp, g1, b1, m1, v1, b_gcn, As, Ad, sw)


def _tc4_body(aggbp, denp, a_s, a_d, skip1, x1, W_gat, bg2, g2, b2, m2, v2,
              W_px1, b_px1, am, W_np1, b_np1, W_np2, b_np2, W_np3, b_np3,
              xf_ref, np_ref):
    ex_self = jnp.exp(_leaky(a_s[...] + a_d[...], 0.2))
    den = jnp.sum(denp[...], axis=0) + ex_self
    sk = skip1[...]
    acc = jnp.zeros((RB, HID), F32)
    for h in range(HEADS):
        agg = aggbp[0, h] + ex_self[:, h:h + 1] * sk
        agg = agg / den[:, h:h + 1]
        acc = acc + jnp.dot(agg, W_gat[:, h * HID:(h + 1) * HID],
                            preferred_element_type=F32)
    x2 = acc * (1.0 / HEADS) + bg2[...]
    x2 = _leaky(_bnk(x2, g2[...], b2[...], m2[...], v2[...]))
    x1p = jnp.dot(x1[...], W_px1[...], preferred_element_type=F32) + b_px1[...]
    a = am[0, 0]
    xf = a * x1p + (1.0 - a) * x2
    h1 = _leaky(jnp.dot(xf, W_np1[...], preferred_element_type=F32)
                + b_np1[...])
    h2 = jnp.dot(h1, W_np2[...], preferred_element_type=F32) + b_np2[...]
    h2 = jnp.logaddexp(h2, 0.0)
    npr = jnp.dot(h2, W_np3[...], preferred_element_type=F32) + b_np3[...]
    xf_ref[...] = xf
    np_ref[...] = jnp.broadcast_to(npr, (RB, 8))


def _tc4(aggb_p, den_p, a_s, a_d, skip1, x1, W_gat, b_gat, g2, b2, m2, v2,
         W_px1, b_px1, am, W_np1, b_np1, W_np2, b_np2, W_np3, b_np3):
    return pl.pallas_call(
        _tc4_body,
        grid=(GRID,),
        in_specs=[pl.BlockSpec((1, HEADS, RB, 128),
                               lambda i: (i // (HNP // RB), 0,
                                          i % (HNP // RB), 0)),
                  _rows(8, lead=(NW,)),
                  _rows(8), _rows(8), _rows(OUTD), _rows(OUTD),
                  _full((IND, HEADS * HID)), _full((1, HID)),
                  _full((1, HID)), _full((1, HID)), _full((1, HID)),
                  _full((1, HID)), _full((OUTD, HID)), _full((1, HID)),
                  _full((1, 1)), _full((HID, HID // 2)),
                  _full((1, HID // 2)), _full((HID // 2, HID // 4)),
                  _full((1, HID // 4)), _full((HID // 4, 1)),
                  _full((1, 1))],
        out_specs=[_rows(HID), _rows(8)],
        out_shape=[jax.ShapeDtypeStruct((NPAD, HID), F32),
                   jax.ShapeDtypeStruct((NPAD, 8), F32)],
    )(aggb_p, den_p, a_s, a_d, skip1, x1, W_gat, b_gat, g2, b2, m2, v2,
      W_px1, b_px1, am, W_np1, b_np1, W_np2, b_np2, W_np3, b_np3)


def kernel(x_in, edge_index, gamma0, beta0, mean0, var0, W_gcn, b_gcn,
           gamma1, beta1, mean1, var1, W_skip, b_skip, W_gat, att_src,
           att_dst, b_gat, gamma2, beta2, mean2, var2, W_px1, b_px1,
           skip_weight, alpha_mix, W_np1, b_np1, W_np2, b_np2, W_np3, b_np3):
    src = edge_index[0]
    dst = edge_index[1]
    r1 = lambda a: a.reshape(1, -1)

    x_pad = jnp.pad(x_in, ((0, NPAD - N_NODES), (0, 0)))

    As, Ad = _tc_prep(W_gat, att_src, att_dst)
    xl, xp = _tc1(x_pad, r1(gamma0), r1(beta0), r1(mean0), r1(var0),
                  W_gcn, W_skip, r1(b_skip))
    deg_p = _sc_deg(dst)
    xls = _tc2(deg_p, xl)
    agg_p = _sc_gcn(src, dst, xls)
    x1, skip1, a_s, a_d = _tc3(deg_p, agg_p, xl, xp, r1(gamma1), r1(beta1),
                               r1(mean1), r1(var1), r1(b_gcn), As, Ad,
                               skip_weight.reshape(1, 1))
    as_flat = a_s[:N_NODES, :HEADS].reshape(-1)
    ad_flat = a_d[:N_NODES, :HEADS].reshape(-1)
    exb = _sc_att(src, dst, as_flat, ad_flat)
    exb2d = exb.reshape(N_EDGES, 16)
    den_p = _sc_den(dst, exb).reshape(NW, NPAD, 8)
    aggb_p = _sc_gat(src, dst, skip1, exb2d)
    xf, npr = _tc4(aggb_p, den_p, a_s, a_d, skip1, x1, W_gat, r1(b_gat),
                   r1(gamma2), r1(beta2), r1(mean2), r1(var2), W_px1,
                   r1(b_px1), alpha_mix.reshape(1, 1), W_np1, r1(b_np1),
                   W_np2, r1(b_np2), W_np3, r1(b_np3))
    return xf[:N_NODES], npr[:N_NODES, 0:1]


# speedup vs baseline: 16.2001x; 1.8599x over previous
import functools
import jax
import jax.numpy as jnp
from jax import lax
from jax.experimental import pallas as pl
from jax.experimental.pallas import tpu as pltpu
from jax.experimental.pallas import tpu_sc as plsc

F32 = jnp.float32

N_NODES = 10000
N_EDGES = 320000
IND = 128
OUTD = 128
HEADS = 5
HID = 2 * OUTD

NPAD = 10240
NC = 2
NS = 16
NW = NC * NS
EPT = N_EDGES // NW
K = 80
ITERS = EPT // K
RPT = NPAD // NS

_sc_mesh = plsc.VectorSubcoreMesh(core_axis_name="c", subcore_axis_name="s")

HNP = NPAD // 2
ACC_R = HNP + 16
ZPT = ACC_R // NS
RPT2 = HNP // NS
EPT2 = N_EDGES // NS
ITERS2 = EPT2 // K


def _zero16(ref, nrows, ncol16):
    z = jnp.zeros((16,), F32)

    def body(j, _):
        for q in range(ncol16):
            ref[j, pl.ds(q * 16, 16)] = z
        return 0

    lax.fori_loop(0, nrows, body, 0)


@functools.partial(
    pl.kernel,
    out_type=jax.ShapeDtypeStruct((NW, NPAD), F32),
    mesh=_sc_mesh,
    scratch_types=[
        pltpu.VMEM((K,), jnp.int32),
        pltpu.VMEM((NPAD,), F32),
    ],
    compiler_params=pltpu.CompilerParams(needs_layout_passes=False),
)
def _sc_deg(dst_hbm, deg_out, didx, deg_t):
    c = lax.axis_index("c")
    s = lax.axis_index("s")
    w = c * NS + s
    z = jnp.zeros((16,), F32)

    def zr(j, _):
        deg_t[pl.ds(j * 16, 16)] = z
        return 0

    lax.fori_loop(0, NPAD // 16, zr, 0)

    ones = jnp.ones((16,), F32)
    woff = w * EPT

    def step(i, _):
        base = pl.multiple_of(woff + i * K, 8)
        pltpu.sync_copy(dst_hbm.at[pl.ds(base, K)], didx)

        def grp(g, _):
            d16 = didx[pl.ds(g * 16, 16)]
            plsc.addupdate_scatter(deg_t, [d16], ones)
            return 0

        lax.fori_loop(0, K // 16, grp, 0)
        return 0

    lax.fori_loop(0, ITERS, step, 0)
    pltpu.sync_copy(deg_t, deg_out.at[w])


def _localize(didx, lidx, c):
    lo = c * HNP

    def body(jj, _):
        v = didx[pl.ds(jj * 16, 16)] - lo
        ok = jnp.logical_and(v >= 0, v < HNP)
        lidx[pl.ds(jj * 16, 16)] = jnp.where(ok, v, HNP)
        return 0

    lax.fori_loop(0, K // 16, body, 0)


@functools.partial(
    pl.kernel,
    out_type=jax.ShapeDtypeStruct((NC, HNP, 128), F32),
    mesh=_sc_mesh,
    scratch_types=[
        pltpu.VMEM((K,), jnp.int32),
        pltpu.VMEM((K,), jnp.int32),
        pltpu.VMEM((K,), jnp.int32),
        pltpu.VMEM((K, 128), F32),
        pltpu.VMEM((K,), jnp.int32),
        pltpu.VMEM((K,), jnp.int32),
        pltpu.VMEM((K,), jnp.int32),
        pltpu.VMEM((K, 128), F32),
        pltpu.VMEM((ZPT, 128), F32),
        pltpu.SemaphoreType.DMA,
        pltpu.SemaphoreType.DMA,
        pltpu.SemaphoreType.DMA,
        pltpu.SemaphoreType.DMA,
        pltpu.SemaphoreType.DMA,
        pltpu.VMEM_SHARED((ACC_R, 128), F32),
    ],
)
def _sc_gcn(src_hbm, dst_hbm, xls_hbm, agg_out, sidx0, didx0, lidx0, rows0,
            sidx1, didx1, lidx1, rows1, stage, isem, gsem0, gsem1, ssem0,
            ssem1, agg_sh):
    c = lax.axis_index("c")
    s = lax.axis_index("s")

    _zero16(stage, ZPT, 8)
    pltpu.sync_copy(stage, agg_sh.at[pl.ds(s * ZPT, ZPT)])
    plsc.subcore_barrier()

    woff = s * EPT2

    def pair(i, _):
        base0 = pl.multiple_of(woff + (i * 2) * K, 8)
        base1 = pl.multiple_of(woff + (i * 2 + 1) * K, 8)
        ds0 = pltpu.async_copy(src_hbm.at[pl.ds(base0, K)], sidx0, isem)
        dd0 = pltpu.async_copy(dst_hbm.at[pl.ds(base0, K)], didx0, isem)
        ds0.wait()
        dd0.wait()
        g0 = pltpu.async_copy(xls_hbm.at[sidx0], rows0, gsem0)
        ds1 = pltpu.async_copy(src_hbm.at[pl.ds(base1, K)], sidx1, isem)
        dd1 = pltpu.async_copy(dst_hbm.at[pl.ds(base1, K)], didx1, isem)
        _localize(didx0, lidx0, c)
        ds1.wait()
        dd1.wait()
        g0.wait()
        g1 = pltpu.async_copy(xls_hbm.at[sidx1], rows1, gsem1)
        s0 = pltpu.async_copy(rows0, agg_sh.at[lidx0], ssem0, add=True)
        _localize(didx1, lidx1, c)
        g1.wait()
        s1 = pltpu.async_copy(rows1, agg_sh.at[lidx1], ssem1, add=True)
        s0.wait()
        s1.wait()
        return 0

    lax.fori_loop(0, ITERS2 // 2, pair, 0)
    plsc.subcore_barrier()

    pltpu.sync_copy(agg_sh.at[pl.ds(s * RPT2, RPT2)],
                    stage.at[pl.ds(0, RPT2)])
    pltpu.sync_copy(stage.at[pl.ds(0, RPT2)],
                    agg_out.at[c, pl.ds(s * RPT2, RPT2)])


@functools.partial(
    pl.kernel,
    out_type=jax.ShapeDtypeStruct((N_EDGES * 16,), F32),
    mesh=_sc_mesh,
    scratch_types=[
        pltpu.VMEM((K,), jnp.int32),
        pltpu.VMEM((K,), jnp.int32),
        pltpu.VMEM((N_NODES * HEADS,), F32),
        pltpu.VMEM((N_NODES * HEADS,), F32),
        pltpu.VMEM((K * 16,), F32),
    ],
    compiler_params=pltpu.CompilerParams(needs_layout_passes=False),
)
def _sc_att(src_hbm, dst_hbm, asf_hbm, adf_hbm, exb_out, sidx, didx,
            as_t, ad_t, exv):
    c = lax.axis_index("c")
    s = lax.axis_index("s")

    pltpu.sync_copy(asf_hbm, as_t)
    pltpu.sync_copy(adf_hbm, ad_t)
    z = jnp.zeros((16,), F32)

    def zr(j, _):
        exv[pl.ds(j * 16, 16)] = z
        return 0

    lax.fori_loop(0, K, zr, 0)

    woff = (c * NS + s) * EPT

    def step(i, _):
        base = pl.multiple_of(woff + i * K, 8)
        pltpu.sync_copy(src_hbm.at[pl.ds(base, K)], sidx)
        pltpu.sync_copy(dst_hbm.at[pl.ds(base, K)], didx)

        def grp(g, _):
            s16 = sidx[pl.ds(g * 16, 16)] * HEADS
            d16 = didx[pl.ds(g * 16, 16)] * HEADS
            eids = (jnp.arange(16, dtype=jnp.int32) + g * 16) * 16
            for h in range(HEADS):
                av = plsc.load_gather(as_t, [s16 + h])
                dv = plsc.load_gather(ad_t, [d16 + h])
                v = av + dv
                v = jnp.where(v > 0, v, 0.2 * v)
                plsc.store_scatter(exv, [eids + h], jnp.exp(v))
            return 0

        lax.fori_loop(0, K // 16, grp, 0)
        pltpu.sync_copy(exv, exb_out.at[pl.ds(base * 16, K * 16)])
        return 0

    lax.fori_loop(0, ITERS, step, 0)


@functools.partial(
    pl.kernel,
    out_type=jax.ShapeDtypeStruct((NW, NPAD * 8), F32),
    mesh=_sc_mesh,
    scratch_types=[
        pltpu.VMEM((K,), jnp.int32),
        pltpu.VMEM((K * 16,), F32),
        pltpu.VMEM((NPAD * 8,), F32),
    ],
    compiler_params=pltpu.CompilerParams(needs_layout_passes=False),
)
def _sc_den(dst_hbm, exb_hbm, den_out, didx, exv, den_t):
    c = lax.axis_index("c")
    s = lax.axis_index("s")
    w = c * NS + s
    z = jnp.zeros((16,), F32)

    def zr(j, _):
        den_t[pl.ds(j * 16, 16)] = z
        return 0

    lax.fori_loop(0, NPAD * 8 // 16, zr, 0)

    woff = w * EPT

    def step(i, _):
        base = pl.multiple_of(woff + i * K, 8)
        pltpu.sync_copy(dst_hbm.at[pl.ds(base, K)], didx)
        pltpu.sync_copy(exb_hbm.at[pl.ds(base * 16, K * 16)], exv)

        def grp(g, _):
            d16 = didx[pl.ds(g * 16, 16)] * 8
            eids = (jnp.arange(16, dtype=jnp.int32) + g * 16) * 16
            for h in range(HEADS):
                exg = plsc.load_gather(exv, [eids + h])
                plsc.addupdate_scatter(den_t, [d16 + h], exg)
            return 0

        lax.fori_loop(0, K // 16, grp, 0)
        return 0

    lax.fori_loop(0, ITERS, step, 0)
    pltpu.sync_copy(den_t, den_out.at[w])


@functools.partial(
    pl.kernel,
    out_type=jax.ShapeDtypeStruct((NC, HEADS, HNP, 128), F32),
    mesh=_sc_mesh,
    scratch_types=[
        pltpu.VMEM((K,), jnp.int32),
        pltpu.VMEM((K,), jnp.int32),
        pltpu.VMEM((K,), jnp.int32),
        pltpu.VMEM((K, 128), F32),
        pltpu.VMEM((K, 128), F32),
        pltpu.VMEM((K * 16,), F32),
        pltpu.VMEM((K,), jnp.int32),
        pltpu.VMEM((K,), jnp.int32),
        pltpu.VMEM((K,), jnp.int32),
        pltpu.VMEM((K, 128), F32),
        pltpu.VMEM((K, 128), F32),
        pltpu.VMEM((K * 16,), F32),
        pltpu.VMEM((ZPT, 128), F32),
        pltpu.SemaphoreType.DMA,
        pltpu.SemaphoreType.DMA,
        pltpu.SemaphoreType.DMA,
        pltpu.SemaphoreType.DMA,
        pltpu.SemaphoreType.DMA,
        pltpu.VMEM_SHARED((ACC_R, 128), F32),
    ],
)
def _sc_gat(src_hbm, dst_hbm, skip1_hbm, exb_hbm, agg_out,
            sidx0, didx0, lidx0, rows0, wrows0, exv0,
            sidx1, didx1, lidx1, rows1, wrows1, exv1,
            stage, isem, gsem0, gsem1, ssem0, ssem1, agg_sh):
    c = lax.axis_index("c")
    s = lax.axis_index("s")
    woff = s * EPT2

    def _wmul(rows_r, exv_r, wrows_r, h):
        def inner(j, _):
            ev = exv_r[pl.ds(j * 16, 16)]
            w = jnp.full((16,), ev[h], F32)
            for q in range(8):
                wrows_r[j, pl.ds(q * 16, 16)] = (
                    rows_r[j, pl.ds(q * 16, 16)] * w)
            return 0

        lax.fori_loop(0, K, inner, 0)

    for h in range(HEADS):
        _zero16(stage, ZPT, 8)
        pltpu.sync_copy(stage, agg_sh.at[pl.ds(s * ZPT, ZPT)])
        plsc.subcore_barrier()

        def pair(i, _):
            base0 = pl.multiple_of(woff + (i * 2) * K, 8)
            base1 = pl.multiple_of(woff + (i * 2 + 1) * K, 8)
            ds0 = pltpu.async_copy(src_hbm.at[pl.ds(base0, K)], sidx0, isem)
            dd0 = pltpu.async_copy(dst_hbm.at[pl.ds(base0, K)], didx0, isem)
            de0 = pltpu.async_copy(exb_hbm.at[pl.ds(base0 * 16, K * 16)],
                                   exv0, isem)
            ds0.wait()
            g0 = pltpu.async_copy(skip1_hbm.at[sidx0], rows0, gsem0)
            ds1 = pltpu.async_copy(src_hbm.at[pl.ds(base1, K)], sidx1, isem)
            dd1 = pltpu.async_copy(dst_hbm.at[pl.ds(base1, K)], didx1, isem)
            de1 = pltpu.async_copy(exb_hbm.at[pl.ds(base1 * 16, K * 16)],
                                   exv1, isem)
            dd0.wait()
            de0.wait()
            ds1.wait()
            g1 = pltpu.async_copy(skip1_hbm.at[sidx1], rows1, gsem1)
            _localize(didx0, lidx0, c)
            g0.wait()
            _wmul(rows0, exv0, wrows0, h)
            s0 = pltpu.async_copy(wrows0, agg_sh.at[lidx0], ssem0, add=True)
            dd1.wait()
            de1.wait()
            _localize(didx1, lidx1, c)
            g1.wait()
            _wmul(rows1, exv1, wrows1, h)
            s1 = pltpu.async_copy(wrows1, agg_sh.at[lidx1], ssem1, add=True)
            s0.wait()
            s1.wait()
            return 0

        lax.fori_loop(0, ITERS2 // 2, pair, 0)
        plsc.subcore_barrier()

        pltpu.sync_copy(agg_sh.at[pl.ds(s * RPT2, RPT2)],
                        stage.at[pl.ds(0, RPT2)])
        pltpu.sync_copy(stage.at[pl.ds(0, RPT2)],
                        agg_out.at[c, h, pl.ds(s * RPT2, RPT2)])
        plsc.subcore_barrier()


RB = 512
GRID = NPAD // RB


def _full(shape):
    return pl.BlockSpec(shape, lambda i: tuple(0 for _ in shape))


def _rows(width, lead=()):
    nlead = len(lead)
    return pl.BlockSpec(lead + (RB, width),
                        lambda i: tuple(0 for _ in range(nlead)) + (i, 0))


def _leaky(x, slope=0.01):
    return jnp.where(x > 0, x, slope * x)


def _bnk(x, g, b, m, v):
    return (x - m) * lax.rsqrt(v + 1e-5) * g + b


def _tc_prep_body(W_gat, att_s, att_d, As_ref, Ad_ref):
    cols_s = []
    cols_d = []
    for h in range(HEADS):
        Wh = W_gat[:, h * HID:(h + 1) * HID]
        cols_s.append(jnp.dot(Wh, att_s[h, :][:, None],
                              preferred_element_type=F32))
        cols_d.append(jnp.dot(Wh, att_d[h, :][:, None],
                              preferred_element_type=F32))
    z = jnp.zeros((IND, 8 - HEADS), F32)
    As_ref[...] = jnp.concatenate(cols_s + [z], axis=1)
    Ad_ref[...] = jnp.concatenate(cols_d + [z], axis=1)


def _tc_prep(W_gat, att_src, att_dst):
    return pl.pallas_call(
        _tc_prep_body,
        grid=(1,),
        in_specs=[_full((IND, HEADS * HID)), _full((HEADS, HID)),
                  _full((HEADS, HID))],
        out_specs=[_full((IND, 8)), _full((IND, 8))],
        out_shape=[jax.ShapeDtypeStruct((IND, 8), F32),
                   jax.ShapeDtypeStruct((IND, 8), F32)],
    )(W_gat, att_src, att_dst)


def _tc1_body(xin, g0, b0, m0, v0, Wg, Ws, bs, xl_ref, xp_ref):
    x = _bnk(xin[...], g0[...], b0[...], m0[...], v0[...])
    xl_ref[...] = jnp.dot(x, Wg[...], preferred_element_type=F32)
    xp_ref[...] = jnp.dot(x, Ws[...], preferred_element_type=F32) + bs[...]


def _tc1(x_in, g0, b0, m0, v0, W_gcn, W_skip, b_skip):
    return pl.pallas_call(
        _tc1_body,
        grid=(GRID,),
        in_specs=[_rows(IND)] + [_full((1, IND))] * 4 +
                 [_full((IND, OUTD)), _full((IND, OUTD)), _full((1, OUTD))],
        out_specs=[_rows(OUTD), _rows(OUTD)],
        out_shape=[jax.ShapeDtypeStruct((NPAD, OUTD), F32),
                   jax.ShapeDtypeStruct((NPAD, OUTD), F32)],
    )(x_in, g0, b0, m0, v0, W_gcn, W_skip, b_skip)


def _tc2_body(degp, xl, xls_ref):
    deg = jnp.sum(degp[...], axis=0)[:, None] + 1.0
    dis = lax.rsqrt(deg)
    xls_ref[...] = dis * xl[...]


def _tc2(deg_p, xl):
    return pl.pallas_call(
        _tc2_body,
        grid=(GRID,),
        in_specs=[pl.BlockSpec((NW, RB), lambda i: (0, i)), _rows(OUTD)],
        out_specs=_rows(OUTD),
        out_shape=jax.ShapeDtypeStruct((NPAD, OUTD), F32),
    )(deg_p, xl)


def _tc3_body(degp, aggp, xl, xp, g1, b1, m1, v1, bg, As, Ad, sw,
              x1_ref, skip1_ref, as_ref, ad_ref):
    deg = jnp.sum(degp[...], axis=0)[:, None] + 1.0
    dis = lax.rsqrt(deg)
    tot = aggp[0] + dis * xl[...]
    x1 = dis * tot + bg[...]
    x1 = _leaky(_bnk(x1, g1[...], b1[...], m1[...], v1[...]))
    skip1 = sw[0, 0] * xp[...] + x1
    x1_ref[...] = x1
    skip1_ref[...] = skip1
    as_ref[...] = jnp.dot(skip1, As[...], preferred_element_type=F32)
    ad_ref[...] = jnp.dot(skip1, Ad[...], preferred_element_type=F32)


def _tc3(deg_p, agg_p, xl, xp, g1, b1, m1, v1, b_gcn, As, Ad, sw):
    return pl.pallas_call(
        _tc3_body,
        grid=(GRID,),
        in_specs=[pl.BlockSpec((NW, RB), lambda i: (0, i)),
                  pl.BlockSpec((1, RB, OUTD),
                               lambda i: (i // (HNP // RB), i % (HNP // RB),
                                          0)),
                  _rows(OUTD), _rows(OUTD)] + [_full((1, OUTD))] * 5 +
                 [_full((OUTD, 8)), _full((OUTD, 8)), _full((1, 1))],
        out_specs=[_rows(OUTD), _rows(OUTD), _rows(8), _rows(8)],
        out_shape=[jax.ShapeDtypeStruct((NPAD, OUTD), F32),
                   jax.ShapeDtypeStruct((NPAD, OUTD), F32),
                   jax.ShapeDtypeStruct((NPAD, 8), F32),
                   jax.ShapeDtypeStruct((NPAD, 8), F32)],
    )(deg_p, agg_p, xl, xp, g1, b1, m1, v1, b_gcn, As, Ad, sw)


def _tc4_body(aggbp, denp, a_s, a_d, skip1, x1, W_gat, bg2, g2, b2, m2, v2,
              W_px1, b_px1, am, W_np1, b_np1, W_np2, b_np2, W_np3, b_np3,
              xf_ref, np_ref):
    ex_self = jnp.exp(_leaky(a_s[...] + a_d[...], 0.2))
    den = jnp.sum(denp[...], axis=0) + ex_self
    sk = skip1[...]
    acc = jnp.zeros((RB, HID), F32)
    for h in range(HEADS):
        agg = aggbp[0, h] + ex_self[:, h:h + 1] * sk
        agg = agg / den[:, h:h + 1]
        acc = acc + jnp.dot(agg, W_gat[:, h * HID:(h + 1) * HID],
                            preferred_element_type=F32)
    x2 = acc * (1.0 / HEADS) + bg2[...]
    x2 = _leaky(_bnk(x2, g2[...], b2[...], m2[...], v2[...]))
    x1p = jnp.dot(x1[...], W_px1[...], preferred_element_type=F32) + b_px1[...]
    a = am[0, 0]
    xf = a * x1p + (1.0 - a) * x2
    h1 = _leaky(jnp.dot(xf, W_np1[...], preferred_element_type=F32)
                + b_np1[...])
    h2 = jnp.dot(h1, W_np2[...], preferred_element_type=F32) + b_np2[...]
    h2 = jnp.logaddexp(h2, 0.0)
    npr = jnp.dot(h2, W_np3[...], preferred_element_type=F32) + b_np3[...]
    xf_ref[...] = xf
    np_ref[...] = jnp.broadcast_to(npr, (RB, 8))


def _tc4(aggb_p, den_p, a_s, a_d, skip1, x1, W_gat, b_gat, g2, b2, m2, v2,
         W_px1, b_px1, am, W_np1, b_np1, W_np2, b_np2, W_np3, b_np3):
    return pl.pallas_call(
        _tc4_body,
        grid=(GRID,),
        in_specs=[pl.BlockSpec((1, HEADS, RB, 128),
                               lambda i: (i // (HNP // RB), 0,
                                          i % (HNP // RB), 0)),
                  _rows(8, lead=(NW,)),
                  _rows(8), _rows(8), _rows(OUTD), _rows(OUTD),
                  _full((IND, HEADS * HID)), _full((1, HID)),
                  _full((1, HID)), _full((1, HID)), _full((1, HID)),
                  _full((1, HID)), _full((OUTD, HID)), _full((1, HID)),
                  _full((1, 1)), _full((HID, HID // 2)),
                  _full((1, HID // 2)), _full((HID // 2, HID // 4)),
                  _full((1, HID // 4)), _full((HID // 4, 1)),
                  _full((1, 1))],
        out_specs=[_rows(HID), _rows(8)],
        out_shape=[jax.ShapeDtypeStruct((NPAD, HID), F32),
                   jax.ShapeDtypeStruct((NPAD, 8), F32)],
    )(aggb_p, den_p, a_s, a_d, skip1, x1, W_gat, b_gat, g2, b2, m2, v2,
      W_px1, b_px1, am, W_np1, b_np1, W_np2, b_np2, W_np3, b_np3)


def kernel(x_in, edge_index, gamma0, beta0, mean0, var0, W_gcn, b_gcn,
           gamma1, beta1, mean1, var1, W_skip, b_skip, W_gat, att_src,
           att_dst, b_gat, gamma2, beta2, mean2, var2, W_px1, b_px1,
           skip_weight, alpha_mix, W_np1, b_np1, W_np2, b_np2, W_np3, b_np3):
    src = edge_index[0]
    dst = edge_index[1]
    r1 = lambda a: a.reshape(1, -1)

    x_pad = jnp.pad(x_in, ((0, NPAD - N_NODES), (0, 0)))

    As, Ad = _tc_prep(W_gat, att_src, att_dst)
    xl, xp = _tc1(x_pad, r1(gamma0), r1(beta0), r1(mean0), r1(var0),
                  W_gcn, W_skip, r1(b_skip))
    deg_p = _sc_deg(dst)
    xls = _tc2(deg_p, xl)
    agg_p = _sc_gcn(src, dst, xls)
    x1, skip1, a_s, a_d = _tc3(deg_p, agg_p, xl, xp, r1(gamma1), r1(beta1),
                               r1(mean1), r1(var1), r1(b_gcn), As, Ad,
                               skip_weight.reshape(1, 1))
    as_flat = a_s[:N_NODES, :HEADS].reshape(-1)
    ad_flat = a_d[:N_NODES, :HEADS].reshape(-1)
    exb = _sc_att(src, dst, as_flat, ad_flat)
    den_p = _sc_den(dst, exb).reshape(NW, NPAD, 8)
    aggb_p = _sc_gat(src, dst, skip1, exb)
    xf, npr = _tc4(aggb_p, den_p, a_s, a_d, skip1, x1, W_gat, r1(b_gat),
                   r1(gamma2), r1(beta2), r1(mean2), r1(var2), W_px1,
                   r1(b_px1), alpha_mix.reshape(1, 1), W_np1, r1(b_np1),
                   W_np2, r1(b_np2), W_np3, r1(b_np3))
    return xf[:N_NODES], npr[:N_NODES, 0:1]


# speedup vs baseline: 17.7587x; 1.0962x over previous
import functools
import jax
import jax.numpy as jnp
from jax import lax
from jax.experimental import pallas as pl
from jax.experimental.pallas import tpu as pltpu
from jax.experimental.pallas import tpu_sc as plsc

F32 = jnp.float32

N_NODES = 10000
N_EDGES = 320000
IND = 128
OUTD = 128
HEADS = 5
HID = 2 * OUTD

NPAD = 10240
NC = 2
NS = 16
NW = NC * NS
EPT = N_EDGES // NW
K = 80
ITERS = EPT // K
RPT = NPAD // NS

_sc_mesh = plsc.VectorSubcoreMesh(core_axis_name="c", subcore_axis_name="s")

HNP = NPAD // 2
ACC_R = HNP + 16
ZPT = ACC_R // NS
RPT2 = HNP // NS
EPT2 = N_EDGES // NS
ITERS2 = EPT2 // K


def _zero16(ref, nrows, ncol16):
    z = jnp.zeros((16,), F32)

    def body(j, _):
        for q in range(ncol16):
            ref[j, pl.ds(q * 16, 16)] = z
        return 0

    lax.fori_loop(0, nrows, body, 0)


@functools.partial(
    pl.kernel,
    out_type=jax.ShapeDtypeStruct((NW, NPAD), F32),
    mesh=_sc_mesh,
    scratch_types=[
        pltpu.VMEM((K,), jnp.int32),
        pltpu.VMEM((NPAD,), F32),
    ],
    compiler_params=pltpu.CompilerParams(needs_layout_passes=False),
)
def _sc_deg(dst_hbm, deg_out, didx, deg_t):
    c = lax.axis_index("c")
    s = lax.axis_index("s")
    w = c * NS + s
    z = jnp.zeros((16,), F32)

    def zr(j, _):
        deg_t[pl.ds(j * 16, 16)] = z
        return 0

    lax.fori_loop(0, NPAD // 16, zr, 0)

    ones = jnp.ones((16,), F32)
    woff = w * EPT

    def step(i, _):
        base = pl.multiple_of(woff + i * K, 8)
        pltpu.sync_copy(dst_hbm.at[pl.ds(base, K)], didx)

        def grp(g, _):
            d16 = didx[pl.ds(g * 16, 16)]
            plsc.addupdate_scatter(deg_t, [d16], ones)
            return 0

        lax.fori_loop(0, K // 16, grp, 0)
        return 0

    lax.fori_loop(0, ITERS, step, 0)
    pltpu.sync_copy(deg_t, deg_out.at[w])


def _localize(didx, lidx, c):
    lo = c * HNP

    def body(jj, _):
        v = didx[pl.ds(jj * 16, 16)] - lo
        ok = jnp.logical_and(v >= 0, v < HNP)
        lidx[pl.ds(jj * 16, 16)] = jnp.where(ok, v, HNP)
        return 0

    lax.fori_loop(0, K // 16, body, 0)


@functools.partial(
    pl.kernel,
    out_type=jax.ShapeDtypeStruct((NC, HNP, 128), F32),
    mesh=_sc_mesh,
    scratch_types=[
        pltpu.VMEM((K,), jnp.int32),
        pltpu.VMEM((K,), jnp.int32),
        pltpu.VMEM((K,), jnp.int32),
        pltpu.VMEM((K, 128), F32),
        pltpu.VMEM((K,), jnp.int32),
        pltpu.VMEM((K,), jnp.int32),
        pltpu.VMEM((K,), jnp.int32),
        pltpu.VMEM((K, 128), F32),
        pltpu.VMEM((ZPT, 128), F32),
        pltpu.SemaphoreType.DMA,
        pltpu.SemaphoreType.DMA,
        pltpu.SemaphoreType.DMA,
        pltpu.SemaphoreType.DMA,
        pltpu.SemaphoreType.DMA,
        pltpu.VMEM_SHARED((ACC_R, 128), F32),
    ],
)
def _sc_gcn(src_hbm, dst_hbm, xls_hbm, agg_out, sidx0, didx0, lidx0, rows0,
            sidx1, didx1, lidx1, rows1, stage, isem, gsem0, gsem1, ssem0,
            ssem1, agg_sh):
    c = lax.axis_index("c")
    s = lax.axis_index("s")

    _zero16(stage, ZPT, 8)
    pltpu.sync_copy(stage, agg_sh.at[pl.ds(s * ZPT, ZPT)])
    plsc.subcore_barrier()

    woff = s * EPT2

    def pair(i, _):
        base0 = pl.multiple_of(woff + (i * 2) * K, 8)
        base1 = pl.multiple_of(woff + (i * 2 + 1) * K, 8)
        ds0 = pltpu.async_copy(src_hbm.at[pl.ds(base0, K)], sidx0, isem)
        dd0 = pltpu.async_copy(dst_hbm.at[pl.ds(base0, K)], didx0, isem)
        ds0.wait()
        dd0.wait()
        g0 = pltpu.async_copy(xls_hbm.at[sidx0], rows0, gsem0)
        ds1 = pltpu.async_copy(src_hbm.at[pl.ds(base1, K)], sidx1, isem)
        dd1 = pltpu.async_copy(dst_hbm.at[pl.ds(base1, K)], didx1, isem)
        _localize(didx0, lidx0, c)
        ds1.wait()
        dd1.wait()
        g0.wait()
        g1 = pltpu.async_copy(xls_hbm.at[sidx1], rows1, gsem1)
        s0 = pltpu.async_copy(rows0, agg_sh.at[lidx0], ssem0, add=True)
        _localize(didx1, lidx1, c)
        g1.wait()
        s1 = pltpu.async_copy(rows1, agg_sh.at[lidx1], ssem1, add=True)
        s0.wait()
        s1.wait()
        return 0

    lax.fori_loop(0, ITERS2 // 2, pair, 0)
    plsc.subcore_barrier()

    pltpu.sync_copy(agg_sh.at[pl.ds(s * RPT2, RPT2)],
                    stage.at[pl.ds(0, RPT2)])
    pltpu.sync_copy(stage.at[pl.ds(0, RPT2)],
                    agg_out.at[c, pl.ds(s * RPT2, RPT2)])


@functools.partial(
    pl.kernel,
    out_type=jax.ShapeDtypeStruct((N_EDGES * 16,), F32),
    mesh=_sc_mesh,
    scratch_types=[
        pltpu.VMEM((K,), jnp.int32),
        pltpu.VMEM((K,), jnp.int32),
        pltpu.VMEM((N_NODES * HEADS,), F32),
        pltpu.VMEM((N_NODES * HEADS,), F32),
        pltpu.VMEM((K * 16,), F32),
    ],
    compiler_params=pltpu.CompilerParams(needs_layout_passes=False),
)
def _sc_att(src_hbm, dst_hbm, asf_hbm, adf_hbm, exb_out, sidx, didx,
            as_t, ad_t, exv):
    c = lax.axis_index("c")
    s = lax.axis_index("s")

    pltpu.sync_copy(asf_hbm, as_t)
    pltpu.sync_copy(adf_hbm, ad_t)
    z = jnp.zeros((16,), F32)

    def zr(j, _):
        exv[pl.ds(j * 16, 16)] = z
        return 0

    lax.fori_loop(0, K, zr, 0)

    woff = (c * NS + s) * EPT

    def step(i, _):
        base = pl.multiple_of(woff + i * K, 8)
        pltpu.sync_copy(src_hbm.at[pl.ds(base, K)], sidx)
        pltpu.sync_copy(dst_hbm.at[pl.ds(base, K)], didx)

        def grp(g, _):
            s16 = sidx[pl.ds(g * 16, 16)] * HEADS
            d16 = didx[pl.ds(g * 16, 16)] * HEADS
            eids = (jnp.arange(16, dtype=jnp.int32) + g * 16) * 16
            for h in range(HEADS):
                av = plsc.load_gather(as_t, [s16 + h])
                dv = plsc.load_gather(ad_t, [d16 + h])
                v = av + dv
                v = jnp.where(v > 0, v, 0.2 * v)
                plsc.store_scatter(exv, [eids + h], jnp.exp(v))
            return 0

        lax.fori_loop(0, K // 16, grp, 0)
        pltpu.sync_copy(exv, exb_out.at[pl.ds(base * 16, K * 16)])
        return 0

    lax.fori_loop(0, ITERS, step, 0)


@functools.partial(
    pl.kernel,
    out_type=jax.ShapeDtypeStruct((NW, NPAD * 8), F32),
    mesh=_sc_mesh,
    scratch_types=[
        pltpu.VMEM((K,), jnp.int32),
        pltpu.VMEM((K * 16,), F32),
        pltpu.VMEM((NPAD * 8,), F32),
    ],
    compiler_params=pltpu.CompilerParams(needs_layout_passes=False),
)
def _sc_den(dst_hbm, exb_hbm, den_out, didx, exv, den_t):
    c = lax.axis_index("c")
    s = lax.axis_index("s")
    w = c * NS + s
    z = jnp.zeros((16,), F32)

    def zr(j, _):
        den_t[pl.ds(j * 16, 16)] = z
        return 0

    lax.fori_loop(0, NPAD * 8 // 16, zr, 0)

    woff = w * EPT

    def step(i, _):
        base = pl.multiple_of(woff + i * K, 8)
        pltpu.sync_copy(dst_hbm.at[pl.ds(base, K)], didx)
        pltpu.sync_copy(exb_hbm.at[pl.ds(base * 16, K * 16)], exv)

        def grp(g, _):
            d16 = didx[pl.ds(g * 16, 16)] * 8
            eids = (jnp.arange(16, dtype=jnp.int32) + g * 16) * 16
            for h in range(HEADS):
                exg = plsc.load_gather(exv, [eids + h])
                plsc.addupdate_scatter(den_t, [d16 + h], exg)
            return 0

        lax.fori_loop(0, K // 16, grp, 0)
        return 0

    lax.fori_loop(0, ITERS, step, 0)
    pltpu.sync_copy(den_t, den_out.at[w])


@functools.partial(
    pl.kernel,
    out_type=jax.ShapeDtypeStruct((NC, HEADS, HNP, 128), F32),
    mesh=_sc_mesh,
    scratch_types=[
        pltpu.VMEM((K,), jnp.int32),
        pltpu.VMEM((K,), jnp.int32),
        pltpu.VMEM((K,), jnp.int32),
        pltpu.VMEM((K, 128), F32),
        pltpu.VMEM((K, 128), F32),
        pltpu.VMEM((K * 16,), F32),
        pltpu.VMEM((K,), jnp.int32),
        pltpu.VMEM((K,), jnp.int32),
        pltpu.VMEM((K,), jnp.int32),
        pltpu.VMEM((K, 128), F32),
        pltpu.VMEM((K, 128), F32),
        pltpu.VMEM((K * 16,), F32),
        pltpu.VMEM((ZPT, 128), F32),
        pltpu.SemaphoreType.DMA,
        pltpu.SemaphoreType.DMA,
        pltpu.SemaphoreType.DMA,
        pltpu.SemaphoreType.DMA,
        pltpu.SemaphoreType.DMA,
        pltpu.VMEM_SHARED((ACC_R, 128), F32),
    ],
)
def _sc_gat(src_hbm, dst_hbm, skip1_hbm, exb_hbm, agg_out,
            sidx0, didx0, lidx0, rows0, wrows0, exv0,
            sidx1, didx1, lidx1, rows1, wrows1, exv1,
            stage, isem, gsem0, gsem1, ssem0, ssem1, agg_sh):
    c = lax.axis_index("c")
    s = lax.axis_index("s")
    woff = s * EPT2

    def _wmul(rows_r, exv_r, wrows_r, h):
        @plsc.parallel_loop(0, K, 1, unroll=4)
        def inner(j):
            ev = exv_r[pl.ds(j * 16, 16)]
            w = jnp.full((16,), ev[h], F32)
            for q in range(8):
                wrows_r[j, pl.ds(q * 16, 16)] = (
                    rows_r[j, pl.ds(q * 16, 16)] * w)

    for h in range(HEADS):
        _zero16(stage, ZPT, 8)
        pltpu.sync_copy(stage, agg_sh.at[pl.ds(s * ZPT, ZPT)])
        plsc.subcore_barrier()

        def pair(i, _):
            base0 = pl.multiple_of(woff + (i * 2) * K, 8)
            base1 = pl.multiple_of(woff + (i * 2 + 1) * K, 8)
            ds0 = pltpu.async_copy(src_hbm.at[pl.ds(base0, K)], sidx0, isem)
            dd0 = pltpu.async_copy(dst_hbm.at[pl.ds(base0, K)], didx0, isem)
            de0 = pltpu.async_copy(exb_hbm.at[pl.ds(base0 * 16, K * 16)],
                                   exv0, isem)
            ds0.wait()
            g0 = pltpu.async_copy(skip1_hbm.at[sidx0], rows0, gsem0)
            ds1 = pltpu.async_copy(src_hbm.at[pl.ds(base1, K)], sidx1, isem)
            dd1 = pltpu.async_copy(dst_hbm.at[pl.ds(base1, K)], didx1, isem)
            de1 = pltpu.async_copy(exb_hbm.at[pl.ds(base1 * 16, K * 16)],
                                   exv1, isem)
            dd0.wait()
            de0.wait()
            ds1.wait()
            g1 = pltpu.async_copy(skip1_hbm.at[sidx1], rows1, gsem1)
            _localize(didx0, lidx0, c)
            g0.wait()
            _wmul(rows0, exv0, wrows0, h)
            s0 = pltpu.async_copy(wrows0, agg_sh.at[lidx0], ssem0, add=True)
            dd1.wait()
            de1.wait()
            _localize(didx1, lidx1, c)
            g1.wait()
            _wmul(rows1, exv1, wrows1, h)
            s1 = pltpu.async_copy(wrows1, agg_sh.at[lidx1], ssem1, add=True)
            s0.wait()
            s1.wait()
            return 0

        lax.fori_loop(0, ITERS2 // 2, pair, 0)
        plsc.subcore_barrier()

        pltpu.sync_copy(agg_sh.at[pl.ds(s * RPT2, RPT2)],
                        stage.at[pl.ds(0, RPT2)])
        pltpu.sync_copy(stage.at[pl.ds(0, RPT2)],
                        agg_out.at[c, h, pl.ds(s * RPT2, RPT2)])
        plsc.subcore_barrier()


RB = 512
GRID = NPAD // RB


def _full(shape):
    return pl.BlockSpec(shape, lambda i: tuple(0 for _ in shape))


def _rows(width, lead=()):
    nlead = len(lead)
    return pl.BlockSpec(lead + (RB, width),
                        lambda i: tuple(0 for _ in range(nlead)) + (i, 0))


def _leaky(x, slope=0.01):
    return jnp.where(x > 0, x, slope * x)


def _bnk(x, g, b, m, v):
    return (x - m) * lax.rsqrt(v + 1e-5) * g + b


def _tc_prep_body(W_gat, att_s, att_d, As_ref, Ad_ref):
    cols_s = []
    cols_d = []
    for h in range(HEADS):
        Wh = W_gat[:, h * HID:(h + 1) * HID]
        cols_s.append(jnp.dot(Wh, att_s[h, :][:, None],
                              preferred_element_type=F32))
        cols_d.append(jnp.dot(Wh, att_d[h, :][:, None],
                              preferred_element_type=F32))
    z = jnp.zeros((IND, 8 - HEADS), F32)
    As_ref[...] = jnp.concatenate(cols_s + [z], axis=1)
    Ad_ref[...] = jnp.concatenate(cols_d + [z], axis=1)


def _tc_prep(W_gat, att_src, att_dst):
    return pl.pallas_call(
        _tc_prep_body,
        grid=(1,),
        in_specs=[_full((IND, HEADS * HID)), _full((HEADS, HID)),
                  _full((HEADS, HID))],
        out_specs=[_full((IND, 8)), _full((IND, 8))],
        out_shape=[jax.ShapeDtypeStruct((IND, 8), F32),
                   jax.ShapeDtypeStruct((IND, 8), F32)],
    )(W_gat, att_src, att_dst)


def _tc1_body(xin, g0, b0, m0, v0, Wg, Ws, bs, xl_ref, xp_ref):
    x = _bnk(xin[...], g0[...], b0[...], m0[...], v0[...])
    xl_ref[...] = jnp.dot(x, Wg[...], preferred_element_type=F32)
    xp_ref[...] = jnp.dot(x, Ws[...], preferred_element_type=F32) + bs[...]


def _tc1(x_in, g0, b0, m0, v0, W_gcn, W_skip, b_skip):
    return pl.pallas_call(
        _tc1_body,
        grid=(GRID,),
        in_specs=[_rows(IND)] + [_full((1, IND))] * 4 +
                 [_full((IND, OUTD)), _full((IND, OUTD)), _full((1, OUTD))],
        out_specs=[_rows(OUTD), _rows(OUTD)],
        out_shape=[jax.ShapeDtypeStruct((NPAD, OUTD), F32),
                   jax.ShapeDtypeStruct((NPAD, OUTD), F32)],
    )(x_in, g0, b0, m0, v0, W_gcn, W_skip, b_skip)


def _tc2_body(degp, xl, xls_ref):
    deg = jnp.sum(degp[...], axis=0)[:, None] + 1.0
    dis = lax.rsqrt(deg)
    xls_ref[...] = dis * xl[...]


def _tc2(deg_p, xl):
    return pl.pallas_call(
        _tc2_body,
        grid=(GRID,),
        in_specs=[pl.BlockSpec((NW, RB), lambda i: (0, i)), _rows(OUTD)],
        out_specs=_rows(OUTD),
        out_shape=jax.ShapeDtypeStruct((NPAD, OUTD), F32),
    )(deg_p, xl)


def _tc3_body(degp, aggp, xl, xp, g1, b1, m1, v1, bg, As, Ad, sw,
              x1_ref, skip1_ref, as_ref, ad_ref):
    deg = jnp.sum(degp[...], axis=0)[:, None] + 1.0
    dis = lax.rsqrt(deg)
    tot = aggp[0] + dis * xl[...]
    x1 = dis * tot + bg[...]
    x1 = _leaky(_bnk(x1, g1[...], b1[...], m1[...], v1[...]))
    skip1 = sw[0, 0] * xp[...] + x1
    x1_ref[...] = x1
    skip1_ref[...] = skip1
    as_ref[...] = jnp.dot(skip1, As[...], preferred_element_type=F32)
    ad_ref[...] = jnp.dot(skip1, Ad[...], preferred_element_type=F32)


def _tc3(deg_p, agg_p, xl, xp, g1, b1, m1, v1, b_gcn, As, Ad, sw):
    return pl.pallas_call(
        _tc3_body,
        grid=(GRID,),
        in_specs=[pl.BlockSpec((NW, RB), lambda i: (0, i)),
                  pl.BlockSpec((1, RB, OUTD),
                               lambda i: (i // (HNP // RB), i % (HNP // RB),
                                          0)),
                  _rows(OUTD), _rows(OUTD)] + [_full((1, OUTD))] * 5 +
                 [_full((OUTD, 8)), _full((OUTD, 8)), _full((1, 1))],
        out_specs=[_rows(OUTD), _rows(OUTD), _rows(8), _rows(8)],
        out_shape=[jax.ShapeDtypeStruct((NPAD, OUTD), F32),
                   jax.ShapeDtypeStruct((NPAD, OUTD), F32),
                   jax.ShapeDtypeStruct((NPAD, 8), F32),
                   jax.ShapeDtypeStruct((NPAD, 8), F32)],
    )(deg_p, agg_p, xl, xp, g1, b1, m1, v1, b_gcn, As, Ad, sw)


def _tc4_body(aggbp, denp, a_s, a_d, skip1, x1, W_gat, bg2, g2, b2, m2, v2,
              W_px1, b_px1, am, W_np1, b_np1, W_np2, b_np2, W_np3, b_np3,
              xf_ref, np_ref):
    ex_self = jnp.exp(_leaky(a_s[...] + a_d[...], 0.2))
    den = jnp.sum(denp[...], axis=0) + ex_self
    sk = skip1[...]
    acc = jnp.zeros((RB, HID), F32)
    for h in range(HEADS):
        agg = aggbp[0, h] + ex_self[:, h:h + 1] * sk
        agg = agg / den[:, h:h + 1]
        acc = acc + jnp.dot(agg, W_gat[:, h * HID:(h + 1) * HID],
                            preferred_element_type=F32)
    x2 = acc * (1.0 / HEADS) + bg2[...]
    x2 = _leaky(_bnk(x2, g2[...], b2[...], m2[...], v2[...]))
    x1p = jnp.dot(x1[...], W_px1[...], preferred_element_type=F32) + b_px1[...]
    a = am[0, 0]
    xf = a * x1p + (1.0 - a) * x2
    h1 = _leaky(jnp.dot(xf, W_np1[...], preferred_element_type=F32)
                + b_np1[...])
    h2 = jnp.dot(h1, W_np2[...], preferred_element_type=F32) + b_np2[...]
    h2 = jnp.logaddexp(h2, 0.0)
    npr = jnp.dot(h2, W_np3[...], preferred_element_type=F32) + b_np3[...]
    xf_ref[...] = xf
    np_ref[...] = jnp.broadcast_to(npr, (RB, 8))


def _tc4(aggb_p, den_p, a_s, a_d, skip1, x1, W_gat, b_gat, g2, b2, m2, v2,
         W_px1, b_px1, am, W_np1, b_np1, W_np2, b_np2, W_np3, b_np3):
    return pl.pallas_call(
        _tc4_body,
        grid=(GRID,),
        in_specs=[pl.BlockSpec((1, HEADS, RB, 128),
                               lambda i: (i // (HNP // RB), 0,
                                          i % (HNP // RB), 0)),
                  _rows(8, lead=(NW,)),
                  _rows(8), _rows(8), _rows(OUTD), _rows(OUTD),
                  _full((IND, HEADS * HID)), _full((1, HID)),
                  _full((1, HID)), _full((1, HID)), _full((1, HID)),
                  _full((1, HID)), _full((OUTD, HID)), _full((1, HID)),
                  _full((1, 1)), _full((HID, HID // 2)),
                  _full((1, HID // 2)), _full((HID // 2, HID // 4)),
                  _full((1, HID // 4)), _full((HID // 4, 1)),
                  _full((1, 1))],
        out_specs=[_rows(HID), _rows(8)],
        out_shape=[jax.ShapeDtypeStruct((NPAD, HID), F32),
                   jax.ShapeDtypeStruct((NPAD, 8), F32)],
    )(aggb_p, den_p, a_s, a_d, skip1, x1, W_gat, b_gat, g2, b2, m2, v2,
      W_px1, b_px1, am, W_np1, b_np1, W_np2, b_np2, W_np3, b_np3)


def kernel(x_in, edge_index, gamma0, beta0, mean0, var0, W_gcn, b_gcn,
           gamma1, beta1, mean1, var1, W_skip, b_skip, W_gat, att_src,
           att_dst, b_gat, gamma2, beta2, mean2, var2, W_px1, b_px1,
           skip_weight, alpha_mix, W_np1, b_np1, W_np2, b_np2, W_np3, b_np3):
    src = edge_index[0]
    dst = edge_index[1]
    r1 = lambda a: a.reshape(1, -1)

    x_pad = jnp.pad(x_in, ((0, NPAD - N_NODES), (0, 0)))

    As, Ad = _tc_prep(W_gat, att_src, att_dst)
    xl, xp = _tc1(x_pad, r1(gamma0), r1(beta0), r1(mean0), r1(var0),
                  W_gcn, W_skip, r1(b_skip))
    deg_p = _sc_deg(dst)
    xls = _tc2(deg_p, xl)
    agg_p = _sc_gcn(src, dst, xls)
    x1, skip1, a_s, a_d = _tc3(deg_p, agg_p, xl, xp, r1(gamma1), r1(beta1),
                               r1(mean1), r1(var1), r1(b_gcn), As, Ad,
                               skip_weight.reshape(1, 1))
    as_flat = a_s[:N_NODES, :HEADS].reshape(-1)
    ad_flat = a_d[:N_NODES, :HEADS].reshape(-1)
    exb = _sc_att(src, dst, as_flat, ad_flat)
    den_p = _sc_den(dst, exb).reshape(NW, NPAD, 8)
    aggb_p = _sc_gat(src, dst, skip1, exb)
    xf, npr = _tc4(aggb_p, den_p, a_s, a_d, skip1, x1, W_gat, r1(b_gat),
                   r1(gamma2), r1(beta2), r1(mean2), r1(var2), W_px1,
                   r1(b_px1), alpha_mix.reshape(1, 1), W_np1, r1(b_np1),
                   W_np2, r1(b_np2), W_np3, r1(b_np3))
    return xf[:N_NODES], npr[:N_NODES, 0:1]
